# initial kernel scaffold (unmeasured)
import jax
import jax.numpy as jnp
from jax import lax
from jax.experimental import pallas as pl
from jax.experimental.pallas import tpu as pltpu


def kernel(
    x,
):
    def body(*refs):
        pass

    out_shape = jax.ShapeDtypeStruct(..., jnp.float32)
    return pl.pallas_call(body, out_shape=out_shape)(...)


# baseline (device time: 49833 ns/iter reference)
import jax
import jax.numpy as jnp
from jax import lax
from jax.experimental import pallas as pl
from jax.experimental.pallas import tpu as pltpu

N_DEV = 8


def _bitonic_sort(x):
    L, _ = x.shape
    i = lax.broadcasted_iota(jnp.int32, (L, 1), 0)
    k = 2
    while k <= L:
        j = k // 2
        while j >= 1:
            up = jnp.roll(x, -j, axis=0)
            dn = jnp.roll(x, j, axis=0)
            is_hi = (i & j) != 0
            asc = (i & k) == 0
            partner = jnp.where(is_hi, dn, up)
            take_min = asc != is_hi
            x = jnp.where(
                take_min, jnp.minimum(x, partner), jnp.maximum(x, partner)
            )
            j //= 2
        k *= 2
    return x


def kernel(x):
    m, n = x.shape

    def body(x_ref, out_ref, gather_ref, send_sems, recv_sems):
        my = lax.axis_index("i")
        left = (my - 1) % N_DEV
        right = (my + 1) % N_DEV

        barrier_sem = pltpu.get_barrier_semaphore()
        for nbr in [left, right]:
            pl.semaphore_signal(
                barrier_sem, inc=1,
                device_id=(nbr,), device_id_type=pl.DeviceIdType.MESH,
            )
        pl.semaphore_wait(barrier_sem, 2)

        gather_ref[0] = x_ref[...].astype(jnp.bfloat16)
        for h in range(N_DEV - 1):
            rdma = pltpu.make_async_remote_copy(
                src_ref=gather_ref.at[h],
                dst_ref=gather_ref.at[h + 1],
                send_sem=send_sems.at[h],
                recv_sem=recv_sems.at[h],
                device_id=(right,),
                device_id_type=pl.DeviceIdType.MESH,
            )
            rdma.start()
            rdma.wait()

        data = gather_ref[...].reshape(N_DEV * m, n)
        data = _bitonic_sort(data)
        gather_ref[...] = data.reshape(N_DEV, m, n)
        out_ref[...] = gather_ref[my].astype(jnp.float32)

    return pl.pallas_call(
        body,
        out_shape=jax.ShapeDtypeStruct((m, n), jnp.float32),
        in_specs=[pl.BlockSpec(memory_space=pltpu.VMEM)],
        out_specs=pl.BlockSpec(memory_space=pltpu.VMEM),
        scratch_shapes=[
            pltpu.VMEM((N_DEV, m, n), jnp.bfloat16),
            pltpu.SemaphoreType.DMA((N_DEV - 1,)),
            pltpu.SemaphoreType.DMA((N_DEV - 1,)),
        ],
        compiler_params=pltpu.CompilerParams(collective_id=0),
    )(x)


# device time: 21986 ns/iter; 2.2666x vs baseline; 2.2666x over previous
import jax
import jax.numpy as jnp
from jax import lax
from jax.experimental import pallas as pl
from jax.experimental.pallas import tpu as pltpu

N_DEV = 8
N_ROUNDS = 3


def _cmpx(x, j, take_min):
    up = jnp.roll(x, -j, axis=0)
    dn = jnp.roll(x, j, axis=0)
    i = lax.broadcasted_iota(jnp.int32, (x.shape[0], 1), 0)
    is_hi = (i & j) != 0
    partner = jnp.where(is_hi, dn, up)
    return jnp.where(take_min, jnp.minimum(x, partner), jnp.maximum(x, partner))


def _bitonic_sort(x, asc):
    L = x.shape[0]
    i = lax.broadcasted_iota(jnp.int32, (L, 1), 0)
    k = 2
    while k <= L:
        j = k // 2
        while j >= 1:
            is_hi = (i & j) != 0
            asc_e = ((i & k) == 0) == asc
            x = _cmpx(x, j, asc_e != is_hi)
            j //= 2
        k *= 2
    return x


def _bitonic_merge(x, asc):
    L = x.shape[0]
    i = lax.broadcasted_iota(jnp.int32, (L, 1), 0)
    j = L // 2
    while j >= 1:
        is_hi = (i & j) != 0
        x = _cmpx(x, j, is_hi != asc)
        j //= 2
    return x


def kernel(x):
    m, n = x.shape

    def body(x_ref, out_ref, work_ref, send_sems, recv_sems):
        my = lax.axis_index("i")

        barrier_sem = pltpu.get_barrier_semaphore()
        for r in range(N_ROUNDS):
            pl.semaphore_signal(
                barrier_sem, inc=1,
                device_id=(my ^ (1 << r),),
                device_id_type=pl.DeviceIdType.MESH,
            )
        pl.semaphore_wait(barrier_sem, N_ROUNDS)

        work_ref[0:m] = _bitonic_sort(x_ref[...].astype(jnp.bfloat16),
                                      (my & 1) == 0)

        for r in range(N_ROUNDS):
            s = m << r
            rdma = pltpu.make_async_remote_copy(
                src_ref=work_ref.at[pl.ds(0, s)],
                dst_ref=work_ref.at[pl.ds(s, s)],
                send_sem=send_sems.at[r],
                recv_sem=recv_sems.at[r],
                device_id=(my ^ (1 << r),),
                device_id_type=pl.DeviceIdType.MESH,
            )
            rdma.start()
            rdma.wait()
            d_next = ((my >> (r + 1)) & 1) == 0
            work_ref[0:2 * s] = _bitonic_merge(work_ref[0:2 * s], d_next)

        out_ref[...] = work_ref[pl.ds(my * m, m)].astype(jnp.float32)

    return pl.pallas_call(
        body,
        out_shape=jax.ShapeDtypeStruct((m, n), jnp.float32),
        in_specs=[pl.BlockSpec(memory_space=pltpu.VMEM)],
        out_specs=pl.BlockSpec(memory_space=pltpu.VMEM),
        scratch_shapes=[
            pltpu.VMEM((N_DEV * m, n), jnp.bfloat16),
            pltpu.SemaphoreType.DMA((N_ROUNDS,)),
            pltpu.SemaphoreType.DMA((N_ROUNDS,)),
        ],
        compiler_params=pltpu.CompilerParams(collective_id=0),
    )(x)


# device time: 19616 ns/iter; 2.5404x vs baseline; 1.1208x over previous
import jax
import jax.numpy as jnp
from jax import lax
from jax.experimental import pallas as pl
from jax.experimental.pallas import tpu as pltpu

N_DEV = 8
N_ROUNDS = 3


def _cmpx_roll(x, j, take_min):
    up = jnp.roll(x, -j, axis=0)
    dn = jnp.roll(x, j, axis=0)
    i = lax.broadcasted_iota(jnp.int32, (x.shape[0], 1), 0)
    is_hi = (i & j) != 0
    partner = jnp.where(is_hi, dn, up)
    return jnp.where(take_min, jnp.minimum(x, partner), jnp.maximum(x, partner))


def _recombine(new_lo, new_hi, L, n):
    return jnp.concatenate(
        [new_lo[:, None], new_hi[:, None]], axis=1
    ).reshape(L, n)


def _bitonic_sort(x, asc):
    L, n = x.shape
    i = lax.broadcasted_iota(jnp.int32, (L, 1), 0)
    k = 2
    while k <= L:
        j = k // 2
        while j >= 1:
            if j >= 8:
                b = L // (2 * j)
                y = x.reshape(b, 2, j, n)
                lo, hi = y[:, 0], y[:, 1]
                mn = jnp.minimum(lo, hi)
                mx = jnp.maximum(lo, hi)
                blk = lax.broadcasted_iota(jnp.int32, (b, 1, 1), 0)
                asc_e = (((blk * 2 * j) & k) == 0) == asc
                x = _recombine(
                    jnp.where(asc_e, mn, mx), jnp.where(asc_e, mx, mn), L, n
                )
            else:
                is_hi = (i & j) != 0
                asc_e = ((i & k) == 0) == asc
                x = _cmpx_roll(x, j, asc_e != is_hi)
            j //= 2
        k *= 2
    return x


def _bitonic_merge(x, asc):
    L, n = x.shape
    i = lax.broadcasted_iota(jnp.int32, (L, 1), 0)
    j = L // 2
    while j >= 1:
        if j >= 8:
            b = L // (2 * j)
            y = x.reshape(b, 2, j, n)
            lo, hi = y[:, 0], y[:, 1]
            mn = jnp.minimum(lo, hi)
            mx = jnp.maximum(lo, hi)
            x = _recombine(
                jnp.where(asc, mn, mx), jnp.where(asc, mx, mn), L, n
            )
        else:
            is_hi = (i & j) != 0
            x = _cmpx_roll(x, j, is_hi != asc)
        j //= 2
    return x


def kernel(x):
    m, n = x.shape

    def body(x_ref, out_ref, work_ref, send_sems, recv_sems):
        my = lax.axis_index("i")

        barrier_sem = pltpu.get_barrier_semaphore()
        for r in range(N_ROUNDS):
            pl.semaphore_signal(
                barrier_sem, inc=1,
                device_id=(my ^ (1 << r),),
                device_id_type=pl.DeviceIdType.MESH,
            )
        pl.semaphore_wait(barrier_sem, N_ROUNDS)

        work_ref[0:m] = _bitonic_sort(x_ref[...].astype(jnp.bfloat16),
                                      (my & 1) == 0)

        for r in range(N_ROUNDS):
            s = m << r
            rdma = pltpu.make_async_remote_copy(
                src_ref=work_ref.at[pl.ds(0, s)],
                dst_ref=work_ref.at[pl.ds(s, s)],
                send_sem=send_sems.at[r],
                recv_sem=recv_sems.at[r],
                device_id=(my ^ (1 << r),),
                device_id_type=pl.DeviceIdType.MESH,
            )
            rdma.start()
            rdma.wait()
            if r < N_ROUNDS - 1:
                d_next = ((my >> (r + 1)) & 1) == 0
                work_ref[0:2 * s] = _bitonic_merge(work_ref[0:2 * s], d_next)

        cur = work_ref[0:8 * m]
        for lvl in range(3):
            half = cur.shape[0] // 2
            a, b = cur[:half], cur[half:]
            keep_lo = ((my >> (2 - lvl)) & 1) == 0
            cur = jnp.where(keep_lo, jnp.minimum(a, b), jnp.maximum(a, b))
        out_ref[...] = _bitonic_merge(cur, True).astype(jnp.float32)

    return pl.pallas_call(
        body,
        out_shape=jax.ShapeDtypeStruct((m, n), jnp.float32),
        in_specs=[pl.BlockSpec(memory_space=pltpu.VMEM)],
        out_specs=pl.BlockSpec(memory_space=pltpu.VMEM),
        scratch_shapes=[
            pltpu.VMEM((N_DEV * m, n), jnp.bfloat16),
            pltpu.SemaphoreType.DMA((N_ROUNDS,)),
            pltpu.SemaphoreType.DMA((N_ROUNDS,)),
        ],
        compiler_params=pltpu.CompilerParams(collective_id=0),
    )(x)


# device time: 13403 ns/iter; 3.7180x vs baseline; 1.4636x over previous
import jax
import jax.numpy as jnp
from jax import lax
from jax.experimental import pallas as pl
from jax.experimental.pallas import tpu as pltpu

N_DEV = 8


def _cmpx_roll(x, j, take_min):
    up = jnp.roll(x, -j, axis=0)
    dn = jnp.roll(x, j, axis=0)
    i = lax.broadcasted_iota(jnp.int32, (x.shape[0], 1), 0)
    is_hi = (i & j) != 0
    partner = jnp.where(is_hi, dn, up)
    return jnp.where(take_min, jnp.minimum(x, partner), jnp.maximum(x, partner))


def _recombine(new_lo, new_hi, L, n):
    return jnp.concatenate(
        [new_lo[:, None], new_hi[:, None]], axis=1
    ).reshape(L, n)


def _bitonic_sort(x, asc):
    L, n = x.shape
    i = lax.broadcasted_iota(jnp.int32, (L, 1), 0)
    k = 2
    while k <= L:
        j = k // 2
        while j >= 1:
            if j >= 8:
                b = L // (2 * j)
                y = x.reshape(b, 2, j, n)
                lo, hi = y[:, 0], y[:, 1]
                mn = jnp.minimum(lo, hi)
                mx = jnp.maximum(lo, hi)
                blk = lax.broadcasted_iota(jnp.int32, (b, 1, 1), 0)
                asc_e = (((blk * 2 * j) & k) == 0) == asc
                x = _recombine(
                    jnp.where(asc_e, mn, mx), jnp.where(asc_e, mx, mn), L, n
                )
            else:
                is_hi = (i & j) != 0
                asc_e = ((i & k) == 0) == asc
                x = _cmpx_roll(x, j, asc_e != is_hi)
            j //= 2
        k *= 2
    return x


def _bitonic_merge(x, asc: bool):
    L, n = x.shape
    i = lax.broadcasted_iota(jnp.int32, (L, 1), 0)
    j = L // 2
    while j >= 1:
        if j >= 8:
            b = L // (2 * j)
            y = x.reshape(b, 2, j, n)
            lo, hi = y[:, 0], y[:, 1]
            mn = jnp.minimum(lo, hi)
            mx = jnp.maximum(lo, hi)
            if asc:
                x = _recombine(mn, mx, L, n)
            else:
                x = _recombine(mx, mn, L, n)
        else:
            is_hi = (i & j) != 0
            x = _cmpx_roll(x, j, is_hi != asc)
        j //= 2
    return x


def kernel(x):
    m, n = x.shape

    def body(x_ref, out_ref, gather_ref, work_ref, send_sems, recv_sems):
        my = lax.axis_index("i")

        barrier_sem = pltpu.get_barrier_semaphore()
        for t in range(1, N_DEV):
            pl.semaphore_signal(
                barrier_sem, inc=1,
                device_id=(my ^ t,),
                device_id_type=pl.DeviceIdType.MESH,
            )
        pl.semaphore_wait(barrier_sem, N_DEV - 1)

        gather_ref[0:m] = _bitonic_sort(x_ref[...].astype(jnp.bfloat16),
                                        (my & 1) == 0)

        rdmas = [None] * N_DEV
        for t in range(1, N_DEV):
            rdmas[t] = pltpu.make_async_remote_copy(
                src_ref=gather_ref.at[pl.ds(0, m)],
                dst_ref=gather_ref.at[pl.ds(t * m, m)],
                send_sem=send_sems.at[t - 1],
                recv_sem=recv_sems.at[t - 1],
                device_id=(my ^ t,),
                device_id_type=pl.DeviceIdType.MESH,
            )
            rdmas[t].start()

        for row, ts, asc in ((0, (1,), True), (2 * m, (2, 3), False)):
            for t in ts:
                rdmas[t].wait_recv()
            work_ref[row:row + 2 * m] = _bitonic_merge(
                gather_ref[row:row + 2 * m], asc)

        work_ref[0:4 * m] = _bitonic_merge(work_ref[0:4 * m], True)

        for row, ts, asc in ((4 * m, (4, 5), True), (6 * m, (6, 7), False)):
            for t in ts:
                rdmas[t].wait_recv()
            work_ref[row:row + 2 * m] = _bitonic_merge(
                gather_ref[row:row + 2 * m], asc)

        work_ref[4 * m:8 * m] = _bitonic_merge(work_ref[4 * m:8 * m], False)

        cur = work_ref[0:8 * m]
        for lvl in range(3):
            half = cur.shape[0] // 2
            a, b = cur[:half], cur[half:]
            keep_lo = ((my >> (2 - lvl)) & 1) == 0
            cur = jnp.where(keep_lo, jnp.minimum(a, b), jnp.maximum(a, b))
        out_ref[...] = _bitonic_merge(cur, True).astype(jnp.float32)

        for t in range(1, N_DEV):
            rdmas[t].wait_send()

    return pl.pallas_call(
        body,
        out_shape=jax.ShapeDtypeStruct((m, n), jnp.float32),
        in_specs=[pl.BlockSpec(memory_space=pltpu.VMEM)],
        out_specs=pl.BlockSpec(memory_space=pltpu.VMEM),
        scratch_shapes=[
            pltpu.VMEM((N_DEV * m, n), jnp.bfloat16),
            pltpu.VMEM((N_DEV * m, n), jnp.bfloat16),
            pltpu.SemaphoreType.DMA((N_DEV - 1,)),
            pltpu.SemaphoreType.DMA((N_DEV - 1,)),
        ],
        compiler_params=pltpu.CompilerParams(collective_id=0),
    )(x)
